# baseline (speedup 1.0000x reference)
import functools

import jax
import jax.numpy as jnp
from jax import lax
from jax.experimental import pallas as pl
from jax.experimental.pallas import tpu as pltpu
from jax.experimental.pallas import tpu_sc as plsc

N_LAYERS = 8
D = 2048
ROW_BLOCK = 1024
BLOCKS_PER_LAYER = D // ROW_BLOCK
LANES = 16
CHUNKS = D // LANES
UNROLL = 8
MAX_FINITE_BITS = 0x7F7FFFFF


def _scores_body(w_ref, o_ref):
    b = pl.program_id(1)

    @pl.when(b == 0)
    def _():
        o_ref[...] = jnp.zeros_like(o_ref)

    w = w_ref[0]
    o_ref[...] += jnp.sum(w * w, axis=0)[None, None, :]


_scores_call = pl.pallas_call(
    _scores_body,
    grid=(N_LAYERS, BLOCKS_PER_LAYER),
    in_specs=[pl.BlockSpec((1, ROW_BLOCK, D), lambda i, b: (i, b, 0))],
    out_specs=pl.BlockSpec((1, 1, D), lambda i, b: (i, 0, 0)),
    out_shape=jax.ShapeDtypeStruct((N_LAYERS, 1, D), jnp.float32),
)


@functools.cache
def _build_select_call():
    return functools.partial(
        pl.kernel,
        mesh=plsc.VectorSubcoreMesh(core_axis_name="c", subcore_axis_name="s"),
        out_type=jax.ShapeDtypeStruct((LANES,), jnp.float32),
        scratch_types=[
            pltpu.VMEM((D,), jnp.float32),
            pltpu.VMEM((2 * LANES,), jnp.float32),
            pltpu.SemaphoreType.DMA,
            pltpu.SemaphoreType.DMA,
            pltpu.VMEM((LANES,), jnp.float32),
            pltpu.VMEM((N_LAYERS * LANES,), jnp.float32),
            pltpu.VMEM((LANES,), jnp.float32),
            pltpu.VMEM_SHARED((N_LAYERS * LANES,), jnp.float32),
        ],
        compiler_params=pltpu.CompilerParams(needs_layout_passes=False),
    )(_select_body)


def _select_body(scores_hbm, aux_hbm, out_hbm,
                 scores_v, aux_v, sem1, sem2, row_v, prod_v, out_v, shared):
    c = lax.axis_index("c")
    sid = lax.axis_index("s")
    lane = lax.iota(jnp.int32, LANES)

    @pl.when((c == 0) & (sid < N_LAYERS))
    def _():
        layer = sid
        h1 = pltpu.async_copy(scores_hbm.at[layer], scores_v, sem1)
        h2 = pltpu.async_copy(aux_hbm, aux_v, sem2)
        h2.wait()
        h1.wait()
        mask = lane == layer
        s_vec = aux_v[pl.ds(0, LANES)]
        y_vec = aux_v[pl.ds(LANES, LANES)]
        t_vec = s_vec.astype(jnp.int32)
        k_vec = t_vec + jnp.where(t_vec.astype(jnp.float32) < s_vec, 1, 0)
        k_vec = jnp.minimum(jnp.maximum(k_vec, 0), D)
        k = jnp.sum(jnp.where(mask, k_vec, 0))
        y_i = jnp.sum(jnp.where(mask, y_vec, 0.0))

        def mmbody(jj, mnmx):
            mn, mx = mnmx
            for u in range(UNROLL):
                v = plsc.bitcast(
                    scores_v[pl.ds((jj * UNROLL + u) * LANES, LANES)],
                    jnp.int32)
                mn = jnp.minimum(mn, v)
                mx = jnp.maximum(mx, v)
            return (mn, mx)

        mn_v, mx_v = lax.fori_loop(
            0, CHUNKS // UNROLL, mmbody,
            (jnp.full((LANES,), MAX_FINITE_BITS, jnp.int32),
             jnp.zeros((LANES,), jnp.int32)))

        def count_le(mid):
            mid_vec = jnp.full((LANES,), mid, dtype=jnp.int32)

            def cbody(jj, cnt):
                for u in range(UNROLL):
                    v = plsc.bitcast(
                        scores_v[pl.ds((jj * UNROLL + u) * LANES, LANES)],
                        jnp.int32)
                    cnt = cnt + jnp.where(v <= mid_vec, 1, 0)
                return cnt

            cnt = lax.fori_loop(0, CHUNKS // UNROLL, cbody,
                                jnp.zeros((LANES,), jnp.int32))
            return jnp.sum(cnt)

        def bcond(lohi):
            lo, hi = lohi
            return lo < hi

        def bbody(lohi):
            lo, hi = lohi
            mid = lo + ((hi - lo) >> 1)
            ge = count_le(mid) >= k
            return (jnp.where(ge, lo, mid + 1), jnp.where(ge, mid, hi))

        lo, _ = lax.while_loop(
            bcond, bbody, (jnp.min(mn_v), jnp.max(mx_v)))
        t_bits = jnp.full((LANES,), lo, dtype=jnp.int32)

        def fbody(jj, carry):
            sm, cl = carry
            for u in range(UNROLL):
                sv = scores_v[pl.ds((jj * UNROLL + u) * LANES, LANES)]
                bv = plsc.bitcast(sv, jnp.int32)
                lt = bv < t_bits
                sm = sm + jnp.where(lt, sv, 0.0)
                cl = cl + jnp.where(lt, 1, 0)
            return (sm, cl)

        sm, cl = lax.fori_loop(
            0, CHUNKS // UNROLL, fbody,
            (jnp.zeros((LANES,), jnp.float32), jnp.zeros((LANES,), jnp.int32)))
        sum_lt = jnp.sum(sm)
        cnt_lt = jnp.sum(cl)
        rem_vec = jnp.full((LANES,), k - cnt_lt, dtype=jnp.int32)
        val_vec = (jnp.full((LANES,), sum_lt, dtype=jnp.float32)
                   + rem_vec.astype(jnp.float32)
                   * plsc.bitcast(t_bits, jnp.float32))
        row_v[...] = jnp.where(mask, y_vec * val_vec, 0.0)
        pltpu.sync_copy(row_v, shared.at[pl.ds(layer * LANES, LANES)])

    plsc.subcore_barrier()

    @pl.when((c == 0) & (sid == 0))
    def _():
        pltpu.sync_copy(shared, prod_v)

        def abody(j, acc):
            return acc + prod_v[pl.ds(j * LANES, LANES)]

        acc = lax.fori_loop(0, N_LAYERS, abody,
                            jnp.zeros((LANES,), jnp.float32))
        out_v[...] = jnp.full((LANES,), jnp.sum(acc), dtype=jnp.float32)
        pltpu.sync_copy(out_v, out_hbm)


def kernel(weights, s, y):
    scores = _scores_call(weights).reshape(N_LAYERS, D)
    aux = jnp.zeros((2 * LANES,), jnp.float32)
    aux = aux.at[:N_LAYERS].set(s).at[LANES:LANES + N_LAYERS].set(y)
    out16 = _build_select_call()(scores, aux)
    return out16[0]


# speedup vs baseline: 1.0060x; 1.0060x over previous
import functools

import jax
import jax.numpy as jnp
from jax import lax
from jax.experimental import pallas as pl
from jax.experimental.pallas import tpu as pltpu
from jax.experimental.pallas import tpu_sc as plsc

N_LAYERS = 8
D = 2048
ROW_BLOCK = 1024
BLOCKS_PER_LAYER = D // ROW_BLOCK
LANES = 16
CHUNKS = D // LANES
UNROLL = 8
MAX_FINITE_BITS = 0x7F7FFFFF


def _scores_body(w_ref, o_ref):
    b = pl.program_id(1)

    @pl.when(b == 0)
    def _():
        o_ref[...] = jnp.zeros_like(o_ref)

    w = w_ref[0]
    o_ref[...] += jnp.sum(w * w, axis=0)[None, None, :]


_scores_call = pl.pallas_call(
    _scores_body,
    grid=(N_LAYERS, BLOCKS_PER_LAYER),
    in_specs=[pl.BlockSpec((1, ROW_BLOCK, D), lambda i, b: (i, b, 0))],
    out_specs=pl.BlockSpec((1, 1, D), lambda i, b: (i, 0, 0)),
    out_shape=jax.ShapeDtypeStruct((N_LAYERS, 1, D), jnp.float32),
)


@functools.cache
def _build_select_call():
    return functools.partial(
        pl.kernel,
        mesh=plsc.VectorSubcoreMesh(core_axis_name="c", subcore_axis_name="s"),
        out_type=jax.ShapeDtypeStruct((LANES,), jnp.float32),
        scratch_types=[
            pltpu.VMEM((D,), jnp.float32),
            pltpu.VMEM((2 * LANES,), jnp.float32),
            pltpu.SemaphoreType.DMA,
            pltpu.SemaphoreType.DMA,
            pltpu.VMEM((LANES,), jnp.float32),
            pltpu.VMEM((N_LAYERS * LANES,), jnp.float32),
            pltpu.VMEM((LANES,), jnp.float32),
            pltpu.VMEM_SHARED((N_LAYERS * LANES,), jnp.float32),
        ],
        compiler_params=pltpu.CompilerParams(needs_layout_passes=False),
    )(_select_body)


def _select_body(scores_hbm, aux_hbm, out_hbm,
                 scores_v, aux_v, sem1, sem2, row_v, prod_v, out_v, shared):
    c = lax.axis_index("c")
    sid = lax.axis_index("s")
    lane = lax.iota(jnp.int32, LANES)

    @pl.when((c == 0) & (sid < N_LAYERS))
    def _():
        layer = sid
        h1 = pltpu.async_copy(scores_hbm.at[layer], scores_v, sem1)
        h2 = pltpu.async_copy(aux_hbm, aux_v, sem2)
        h2.wait()
        h1.wait()
        mask = lane == layer
        s_vec = aux_v[pl.ds(0, LANES)]
        y_vec = aux_v[pl.ds(LANES, LANES)]
        t_vec = s_vec.astype(jnp.int32)
        k_vec = t_vec + jnp.where(t_vec.astype(jnp.float32) < s_vec, 1, 0)
        k_vec = jnp.minimum(jnp.maximum(k_vec, 0), D)
        k = jnp.sum(jnp.where(mask, k_vec, 0))

        def mmbody(jj, mnmx):
            mn, mx = mnmx
            for u in range(UNROLL):
                v = plsc.bitcast(
                    scores_v[pl.ds((jj * UNROLL + u) * LANES, LANES)],
                    jnp.int32)
                mn = jnp.minimum(mn, v)
                mx = jnp.maximum(mx, v)
            return (mn, mx)

        mn_v, mx_v = lax.fori_loop(
            0, CHUNKS // UNROLL, mmbody,
            (jnp.full((LANES,), MAX_FINITE_BITS, jnp.int32),
             jnp.zeros((LANES,), jnp.int32)))

        def count_le(mid):
            mid_vec = jnp.full((LANES,), mid, dtype=jnp.int32)

            def cbody(jj, cnt):
                for u in range(UNROLL):
                    v = plsc.bitcast(
                        scores_v[pl.ds((jj * UNROLL + u) * LANES, LANES)],
                        jnp.int32)
                    cnt = cnt + jnp.where(v <= mid_vec, 1, 0)
                return cnt

            cnt = lax.fori_loop(0, CHUNKS // UNROLL, cbody,
                                jnp.zeros((LANES,), jnp.int32))
            return jnp.sum(cnt)

        def bcond(lohi):
            lo, hi = lohi
            return lo < hi

        def bbody(lohi):
            lo, hi = lohi
            mid = lo + ((hi - lo) >> 1)
            ge = count_le(mid) >= k
            return (jnp.where(ge, lo, mid + 1), jnp.where(ge, mid, hi))

        lo, _ = lax.while_loop(
            bcond, bbody, (jnp.min(mn_v), jnp.max(mx_v)))
        t_bits = jnp.full((LANES,), lo, dtype=jnp.int32)

        def fbody(jj, carry):
            sm, cl = carry
            for u in range(UNROLL):
                sv = scores_v[pl.ds((jj * UNROLL + u) * LANES, LANES)]
                bv = plsc.bitcast(sv, jnp.int32)
                lt = bv < t_bits
                sm = sm + jnp.where(lt, sv, 0.0)
                cl = cl + jnp.where(lt, 1, 0)
            return (sm, cl)

        sm, cl = lax.fori_loop(
            0, CHUNKS // UNROLL, fbody,
            (jnp.zeros((LANES,), jnp.float32), jnp.zeros((LANES,), jnp.int32)))
        sum_lt = jnp.sum(sm)
        cnt_lt = jnp.sum(cl)
        rem_vec = jnp.full((LANES,), k - cnt_lt, dtype=jnp.int32)
        val_vec = (jnp.full((LANES,), sum_lt, dtype=jnp.float32)
                   + rem_vec.astype(jnp.float32)
                   * plsc.bitcast(t_bits, jnp.float32))
        row_v[...] = jnp.where(mask, y_vec * val_vec, 0.0)
        pltpu.sync_copy(row_v, shared.at[pl.ds(layer * LANES, LANES)])

    plsc.subcore_barrier()

    @pl.when((c == 0) & (sid == 0))
    def _():
        pltpu.sync_copy(shared, prod_v)

        def abody(j, acc):
            return acc + prod_v[pl.ds(j * LANES, LANES)]

        acc = lax.fori_loop(0, N_LAYERS, abody,
                            jnp.zeros((LANES,), jnp.float32))
        out_v[...] = jnp.full((LANES,), jnp.sum(acc), dtype=jnp.float32)
        pltpu.sync_copy(out_v, out_hbm)


def kernel(weights, s, y):
    scores = _scores_call(weights).reshape(N_LAYERS, D)
    aux = jnp.zeros((2 * LANES,), jnp.float32)
    aux = aux.at[:N_LAYERS].set(s).at[LANES:LANES + N_LAYERS].set(y)
    out16 = _build_select_call()(scores, aux)
    return out16[0]
